# speedup vs baseline: 86.2992x; 86.2992x over previous
import jax
import jax.numpy as jnp
from jax.experimental import pallas as pl


def _body(att_ref, etT_ref, emb_ref, out_ref):
    att = att_ref[0]
    avg = jnp.sum(att, axis=0) * (1.0 / att.shape[0])
    etT = etT_ref[0]
    emb = emb_ref[...]
    T = emb.shape[0]
    D = emb.shape[1]
    out = jnp.zeros((avg.shape[0], D), jnp.float32)
    for t in range(T):
        s_t = jnp.sum(jnp.where(etT == t, avg, 0.0), axis=1)
        out = out + s_t[:, None] * emb[t][None, :]
    out_ref[0] = out


def kernel(attention_weights, edge_type_matrix, embedding_table):
    B, H, N, _ = attention_weights.shape
    T, D = embedding_table.shape
    etT = jnp.swapaxes(edge_type_matrix.astype(jnp.int32), 1, 2)
    BI = 64
    return pl.pallas_call(
        _body,
        grid=(B, N // BI),
        in_specs=[
            pl.BlockSpec((1, H, BI, N), lambda b, i: (b, 0, i, 0)),
            pl.BlockSpec((1, BI, N), lambda b, i: (b, i, 0)),
            pl.BlockSpec((T, D), lambda b, i: (0, 0)),
        ],
        out_specs=pl.BlockSpec((1, BI, D), lambda b, i: (b, i, 0)),
        out_shape=jax.ShapeDtypeStruct((B, N, D), jnp.float32),
    )(attention_weights, etT, embedding_table)
